# baseline (speedup 1.0000x reference)
import functools

import jax
import jax.numpy as jnp
from jax import lax
from jax.experimental import pallas as pl
from jax.experimental.pallas import tpu as pltpu
from jax.experimental.pallas import tpu_sc as plsc

B = 16384
K = 50
V = 1000000
D = 32
H = 250

NC = 2
NS = 16
NW = NC * NS
BPW = B // NW
RB = 2
RIDX = 2 * K + 4
NR = BPW // RB
NBUF = 8
NI = NR // NBUF
KP = 56


def _sc_stage_body(x_hbm, w_hbm, idx_o, w_o, xbuf, wbuf, idx_buf, w_buf2):
  wid = lax.axis_index("s") * NC + lax.axis_index("c")
  base_b = wid * BPW
  pltpu.sync_copy(x_hbm.at[pl.ds(0, K), pl.ds(base_b, BPW)], xbuf)
  pltpu.sync_copy(w_hbm.at[pl.ds(0, K), pl.ds(base_b, BPW)], wbuf)

  iota = lax.iota(jnp.int32, 16)
  pad_base = iota * 1021 + wid * 37

  def zero_pad(r, _):
    idx_buf[r, pl.ds(RIDX - 16, 16)] = pad_base + r * 53
    return 0

  lax.fori_loop(0, NR, zero_pad, 0)
  colpar = (iota & 1) * K

  def fill_k(k, _):
    for b16 in range(BPW // 16):
      rows = 8 * b16 + (iota >> 1)
      cols = colpar + k
      v = xbuf[k, pl.ds(16 * b16, 16)]
      p = ((v & ~(TRB - 1)) | ((v & (TRB // 4 - 1)) << 2)
           | ((v & (TRB - 1)) >> 11))
      plsc.store_scatter(idx_buf, [rows, cols], p)
      w = wbuf[k, pl.ds(16 * b16, 16)]
      plsc.store_scatter(w_buf2, [rows, cols], w)
    return 0

  lax.fori_loop(0, K, fill_k, 0)

  pltpu.sync_copy(idx_buf, idx_o.at[pl.ds(wid * NR, NR), :])
  pltpu.sync_copy(w_buf2, w_o.at[pl.ds(wid * NR, NR), :])


@jax.jit
def _sc_stage(xt_pad, wt_pad):
  mesh = plsc.VectorSubcoreMesh(core_axis_name="c", subcore_axis_name="s",
                                num_cores=NC, num_subcores=NS)
  f = pl.kernel(
      _sc_stage_body,
      out_type=(jax.ShapeDtypeStruct((NW * NR, RIDX), jnp.int32),
                jax.ShapeDtypeStruct((NW * NR, RIDX), jnp.float32)),
      mesh=mesh,
      compiler_params=pltpu.CompilerParams(use_tc_tiling_on_sc=False,
                                           needs_layout_passes=False),
      scratch_types=[
          pltpu.VMEM((K, BPW), jnp.int32),
          pltpu.VMEM((K, BPW), jnp.float32),
          pltpu.VMEM((NR, RIDX), jnp.int32),
          pltpu.VMEM((NR, RIDX), jnp.float32),
      ],
  )
  return f(xt_pad, wt_pad)


def _sc_bag_body(emb_hbm, x_hbm, w_hbm, out_hbm,
                 idx_all, w_all, rows0, rows1, rows2, rows3,
                 rows4, rows5, rows6, rows7, out_buf,
                 sem0, sem1, sem2, sem3, sem4, sem5, sem6, sem7,
                 sem_i, sem_w):
  wid = lax.axis_index("s") * NC + lax.axis_index("c")
  base_b = wid * BPW
  bufs = (rows0, rows1, rows2, rows3, rows4, rows5, rows6, rows7)
  sems = (sem0, sem1, sem2, sem3, sem4, sem5, sem6, sem7)

  cp_i = pltpu.async_copy(x_hbm.at[pl.ds(wid * NR, NR), :], idx_all, sem_i)
  cp_w = pltpu.async_copy(w_hbm.at[pl.ds(wid * NR, NR), :], w_all, sem_w)
  cp_i.wait()

  def start_gather(r, buf, sem):
    pltpu.async_copy(emb_hbm.at[idx_all.at[r]], buf, sem)

  for s in range(NBUF):
    start_gather(s, bufs[s], sems[s])
  cp_w.wait()

  lane_consts = [jnp.full((16,), lane, jnp.int32) for lane in range(16)]

  def lane_bcast(vec, lane):
    return lax.gather(
        vec, lane_consts[lane][:, None],
        lax.GatherDimensionNumbers(offset_dims=(), collapsed_slice_dims=(0,),
                                   start_index_map=(0,)),
        (1,), mode=lax.GatherScatterMode.PROMISE_IN_BOUNDS)

  def compute_round(r, rows):
    for j in range(RB):
      b_loc = r * RB + j
      cb = j * K
      wvecs = [w_all[r, pl.ds(cb + off, 16)] for off in (0, 16, 32, 34)]
      acc = [[jnp.zeros((16,), jnp.float32) for _ in range(2)]
             for _ in range(5)]
      for k in range(K):
        g = k % 5
        row = j * K + k
        if k < 48:
          wv = lane_bcast(wvecs[k // 16], k % 16)
        else:
          wv = lane_bcast(wvecs[3], k - 34)
        acc[g][0] = acc[g][0] + wv * rows[row, pl.ds(0, 16)]
        acc[g][1] = acc[g][1] + wv * rows[row, pl.ds(16, 16)]
      lo = ((acc[0][0] + acc[1][0]) + (acc[2][0] + acc[3][0])) + acc[4][0]
      hi = ((acc[0][1] + acc[1][1]) + (acc[2][1] + acc[3][1])) + acc[4][1]
      out_buf[b_loc, pl.ds(0, 16)] = lo
      out_buf[b_loc, pl.ds(16, 16)] = hi

  def loop_body(i, _):
    r0 = i * NBUF
    for s in range(NBUF):
      r = r0 + s
      pltpu.make_async_copy(emb_hbm.at[idx_all.at[r]], bufs[s],
                            sems[s]).wait()
      compute_round(r, bufs[s])

      @pl.when(r + NBUF < NR)
      def _():
        start_gather(r + NBUF, bufs[s], sems[s])

    return 0

  lax.fori_loop(0, NI, loop_body, 0)

  pltpu.sync_copy(out_buf, out_hbm.at[pl.ds(base_b, BPW), :])


@jax.jit
def _sc_bag(emb, x2, w_flat):
  mesh = plsc.VectorSubcoreMesh(core_axis_name="c", subcore_axis_name="s",
                                num_cores=NC, num_subcores=NS)
  f = pl.kernel(
      _sc_bag_body,
      out_type=jax.ShapeDtypeStruct((B, D), jnp.float32),
      mesh=mesh,
      compiler_params=pltpu.CompilerParams(use_tc_tiling_on_sc=False),
      scratch_types=[
          pltpu.VMEM((NR, RIDX), jnp.int32),
          pltpu.VMEM((NR, RIDX), jnp.float32),
          pltpu.VMEM((RIDX, D), jnp.float32),
          pltpu.VMEM((RIDX, D), jnp.float32),
          pltpu.VMEM((RIDX, D), jnp.float32),
          pltpu.VMEM((RIDX, D), jnp.float32),
          pltpu.VMEM((RIDX, D), jnp.float32),
          pltpu.VMEM((RIDX, D), jnp.float32),
          pltpu.VMEM((RIDX, D), jnp.float32),
          pltpu.VMEM((RIDX, D), jnp.float32),
          pltpu.VMEM((BPW, D), jnp.float32),
          pltpu.SemaphoreType.DMA,
          pltpu.SemaphoreType.DMA,
          pltpu.SemaphoreType.DMA,
          pltpu.SemaphoreType.DMA,
          pltpu.SemaphoreType.DMA,
          pltpu.SemaphoreType.DMA,
          pltpu.SemaphoreType.DMA,
          pltpu.SemaphoreType.DMA,
          pltpu.SemaphoreType.DMA,
          pltpu.SemaphoreType.DMA,
      ],
  )
  return f(emb, x2, w_flat)


TRB = 8192
TRBM = 3
NTRB = (V + TRB - 1) // TRB
VP = NTRB * TRB


def _tr_body(in_ref, out_ref):
  for m in range(TRBM):
    parts = [in_ref[:, pl.ds(m * TRB + a * (TRB // 4), TRB // 4)]
             for a in range(4)]
    out_ref[pl.ds(m * (TRB // 4), TRB // 4), :] = (
        jnp.concatenate(parts, axis=0).T)


@jax.jit
def _relayout(emb_t):
  assert NTRB % TRBM == 0
  return pl.pallas_call(
      _tr_body,
      grid=(NTRB // TRBM,),
      in_specs=[pl.BlockSpec((D, TRBM * TRB), lambda i: (0, i))],
      out_specs=pl.BlockSpec((TRBM * TRB // 4, 4 * D), lambda i: (i, 0)),
      out_shape=jax.ShapeDtypeStruct((VP // 4, 4 * D), jnp.float32),
  )(emb_t)


def _mlp_body(mean_ref, w1_ref, b1_ref, w2_ref, b2_ref, out_ref):
  w1 = w1_ref[:] * (1.0 / K)
  h = jnp.dot(mean_ref[:], w1, preferred_element_type=jnp.float32)
  h = jnp.maximum(h + b1_ref[:], 0.0)
  out_ref[:] = jnp.dot(h, w2_ref[:], preferred_element_type=jnp.float32) \
      + b2_ref[:]


@jax.jit
def _mlp(mean, W1, b1, W2, b2):
  M = 4096
  grid = (B // M,)
  return pl.pallas_call(
      _mlp_body,
      grid=grid,
      in_specs=[
          pl.BlockSpec((M, D), lambda i: (i, 0)),
          pl.BlockSpec((D, H), lambda i: (0, 0)),
          pl.BlockSpec((1, H), lambda i: (0, 0)),
          pl.BlockSpec((H, 2), lambda i: (0, 0)),
          pl.BlockSpec((1, 2), lambda i: (0, 0)),
      ],
      out_specs=pl.BlockSpec((M, 2), lambda i: (i, 0)),
      out_shape=jax.ShapeDtypeStruct((B, 2), jnp.float32),
  )(mean, W1, b1, W2, b2)


def kernel(x, T, emb, W1, b1, W2, b2):
  xt_pad = jnp.pad(x.astype(jnp.int32).T, ((0, KP - K), (0, 0)))
  wt_pad = jnp.pad(jnp.transpose(T, (2, 1, 0)).reshape(K, B),
                   ((0, KP - K), (0, 0)))
  idx_o, w_o = _sc_stage(xt_pad, wt_pad)
  emb_rows = _relayout(emb.T).reshape(VP, D)
  mean_sum = _sc_bag(emb_rows, idx_o, w_o)
  return _mlp(mean_sum, W1, b1.reshape(1, H), W2, b2.reshape(1, 2))


# speedup vs baseline: 1.0248x; 1.0248x over previous
import functools

import jax
import jax.numpy as jnp
from jax import lax
from jax.experimental import pallas as pl
from jax.experimental.pallas import tpu as pltpu
from jax.experimental.pallas import tpu_sc as plsc

B = 16384
K = 50
V = 1000000
D = 32
H = 250

NC = 2
NS = 16
NW = NC * NS
BPW = B // NW
RB = 4
RIDX = RB * K + 8
NR = BPW // RB
NBUF = 2
NI = NR // NBUF
KP = 56


def _sc_stage_body(x_hbm, w_hbm, idx_o, w_o, xbuf, wbuf, idx_buf, w_buf2):
  wid = lax.axis_index("s") * NC + lax.axis_index("c")
  base_b = wid * BPW
  pltpu.sync_copy(x_hbm.at[pl.ds(0, K), pl.ds(base_b, BPW)], xbuf)
  pltpu.sync_copy(w_hbm.at[pl.ds(0, K), pl.ds(base_b, BPW)], wbuf)

  iota = lax.iota(jnp.int32, 16)
  pad_base = iota * 1021 + wid * 37

  def zero_pad(r, _):
    idx_buf[r, pl.ds(RIDX - 16, 16)] = pad_base + r * 53
    return 0

  lax.fori_loop(0, NR, zero_pad, 0)
  colpar = (iota & 3) * K

  def fill_k(k, _):
    for b16 in range(BPW // 16):
      rows = 4 * b16 + (iota >> 2)
      cols = colpar + k
      v = xbuf[k, pl.ds(16 * b16, 16)]
      p = ((v & ~(TRB - 1)) | ((v & (TRB // 4 - 1)) << 2)
           | ((v & (TRB - 1)) >> 11))
      plsc.store_scatter(idx_buf, [rows, cols], p)
      w = wbuf[k, pl.ds(16 * b16, 16)]
      plsc.store_scatter(w_buf2, [rows, cols], w)
    return 0

  lax.fori_loop(0, K, fill_k, 0)

  pltpu.sync_copy(idx_buf, idx_o.at[pl.ds(wid * NR, NR), :])
  pltpu.sync_copy(w_buf2, w_o.at[pl.ds(wid * NR, NR), :])


@jax.jit
def _sc_stage(xt_pad, wt_pad):
  mesh = plsc.VectorSubcoreMesh(core_axis_name="c", subcore_axis_name="s",
                                num_cores=NC, num_subcores=NS)
  f = pl.kernel(
      _sc_stage_body,
      out_type=(jax.ShapeDtypeStruct((NW * NR, RIDX), jnp.int32),
                jax.ShapeDtypeStruct((NW * NR, RIDX), jnp.float32)),
      mesh=mesh,
      compiler_params=pltpu.CompilerParams(use_tc_tiling_on_sc=False,
                                           needs_layout_passes=False),
      scratch_types=[
          pltpu.VMEM((K, BPW), jnp.int32),
          pltpu.VMEM((K, BPW), jnp.float32),
          pltpu.VMEM((NR, RIDX), jnp.int32),
          pltpu.VMEM((NR, RIDX), jnp.float32),
      ],
  )
  return f(xt_pad, wt_pad)


def _sc_bag_body(emb_hbm, x_hbm, w_hbm, out_hbm,
                 idx_all, w_all, rows0, rows1, out_buf,
                 sem0, sem1, sem_i, sem_w):
  wid = lax.axis_index("s") * NC + lax.axis_index("c")
  base_b = wid * BPW
  bufs = (rows0, rows1)
  sems = (sem0, sem1)

  cp_i = pltpu.async_copy(x_hbm.at[pl.ds(wid * NR, NR), :], idx_all, sem_i)
  cp_w = pltpu.async_copy(w_hbm.at[pl.ds(wid * NR, NR), :], w_all, sem_w)
  cp_i.wait()

  def start_gather(r, buf, sem):
    pltpu.async_copy(emb_hbm.at[idx_all.at[r]], buf, sem)

  for s in range(NBUF):
    start_gather(s, bufs[s], sems[s])
  cp_w.wait()

  lane_consts = [jnp.full((16,), lane, jnp.int32) for lane in range(16)]

  def lane_bcast(vec, lane):
    return lax.gather(
        vec, lane_consts[lane][:, None],
        lax.GatherDimensionNumbers(offset_dims=(), collapsed_slice_dims=(0,),
                                   start_index_map=(0,)),
        (1,), mode=lax.GatherScatterMode.PROMISE_IN_BOUNDS)

  def compute_round(r, rows):
    for j in range(RB):
      b_loc = r * RB + j
      cb = j * K
      wvecs = [w_all[r, pl.ds(cb + off, 16)] for off in (0, 16, 32, 34)]
      acc = [[jnp.zeros((16,), jnp.float32) for _ in range(2)]
             for _ in range(5)]
      for k in range(K):
        g = k % 5
        row = j * K + k
        if k < 48:
          wv = lane_bcast(wvecs[k // 16], k % 16)
        else:
          wv = lane_bcast(wvecs[3], k - 34)
        acc[g][0] = acc[g][0] + wv * rows[row, pl.ds(0, 16)]
        acc[g][1] = acc[g][1] + wv * rows[row, pl.ds(16, 16)]
      lo = ((acc[0][0] + acc[1][0]) + (acc[2][0] + acc[3][0])) + acc[4][0]
      hi = ((acc[0][1] + acc[1][1]) + (acc[2][1] + acc[3][1])) + acc[4][1]
      out_buf[b_loc, pl.ds(0, 16)] = lo
      out_buf[b_loc, pl.ds(16, 16)] = hi

  def loop_body(i, _):
    r0 = i * NBUF
    for s in range(NBUF):
      r = r0 + s
      pltpu.make_async_copy(emb_hbm.at[idx_all.at[r]], bufs[s],
                            sems[s]).wait()
      compute_round(r, bufs[s])

      @pl.when(r + NBUF < NR)
      def _():
        start_gather(r + NBUF, bufs[s], sems[s])

    return 0

  lax.fori_loop(0, NI, loop_body, 0)

  pltpu.sync_copy(out_buf, out_hbm.at[pl.ds(base_b, BPW), :])


@jax.jit
def _sc_bag(emb, x2, w_flat):
  mesh = plsc.VectorSubcoreMesh(core_axis_name="c", subcore_axis_name="s",
                                num_cores=NC, num_subcores=NS)
  f = pl.kernel(
      _sc_bag_body,
      out_type=jax.ShapeDtypeStruct((B, D), jnp.float32),
      mesh=mesh,
      compiler_params=pltpu.CompilerParams(use_tc_tiling_on_sc=False),
      scratch_types=[
          pltpu.VMEM((NR, RIDX), jnp.int32),
          pltpu.VMEM((NR, RIDX), jnp.float32),
          pltpu.VMEM((RIDX, D), jnp.float32),
          pltpu.VMEM((RIDX, D), jnp.float32),
          pltpu.VMEM((BPW, D), jnp.float32),
          pltpu.SemaphoreType.DMA,
          pltpu.SemaphoreType.DMA,
          pltpu.SemaphoreType.DMA,
          pltpu.SemaphoreType.DMA,
      ],
  )
  return f(emb, x2, w_flat)


TRB = 8192
TRBM = 3
NTRB = (V + TRB - 1) // TRB
VP = NTRB * TRB


def _tr_body(in_ref, out_ref):
  for m in range(TRBM):
    parts = [in_ref[:, pl.ds(m * TRB + a * (TRB // 4), TRB // 4)]
             for a in range(4)]
    out_ref[pl.ds(m * (TRB // 4), TRB // 4), :] = (
        jnp.concatenate(parts, axis=0).T)


@jax.jit
def _relayout(emb_t):
  assert NTRB % TRBM == 0
  return pl.pallas_call(
      _tr_body,
      grid=(NTRB // TRBM,),
      in_specs=[pl.BlockSpec((D, TRBM * TRB), lambda i: (0, i))],
      out_specs=pl.BlockSpec((TRBM * TRB // 4, 4 * D), lambda i: (i, 0)),
      out_shape=jax.ShapeDtypeStruct((VP // 4, 4 * D), jnp.float32),
  )(emb_t)


def _mlp_body(mean_ref, w1_ref, b1_ref, w2_ref, b2_ref, out_ref):
  w1 = w1_ref[:] * (1.0 / K)
  h = jnp.dot(mean_ref[:], w1, preferred_element_type=jnp.float32)
  h = jnp.maximum(h + b1_ref[:], 0.0)
  out_ref[:] = jnp.dot(h, w2_ref[:], preferred_element_type=jnp.float32) \
      + b2_ref[:]


@jax.jit
def _mlp(mean, W1, b1, W2, b2):
  M = 2048
  grid = (B // M,)
  return pl.pallas_call(
      _mlp_body,
      grid=grid,
      in_specs=[
          pl.BlockSpec((M, D), lambda i: (i, 0)),
          pl.BlockSpec((D, H), lambda i: (0, 0)),
          pl.BlockSpec((1, H), lambda i: (0, 0)),
          pl.BlockSpec((H, 2), lambda i: (0, 0)),
          pl.BlockSpec((1, 2), lambda i: (0, 0)),
      ],
      out_specs=pl.BlockSpec((M, 2), lambda i: (i, 0)),
      out_shape=jax.ShapeDtypeStruct((B, 2), jnp.float32),
  )(mean, W1, b1, W2, b2)


def kernel(x, T, emb, W1, b1, W2, b2):
  xt_pad = jnp.pad(x.astype(jnp.int32).T, ((0, KP - K), (0, 0)))
  wt_pad = jnp.pad(jnp.transpose(T, (2, 1, 0)).reshape(K, B),
                   ((0, KP - K), (0, 0)))
  idx_o, w_o = _sc_stage(xt_pad, wt_pad)
  emb_rows = _relayout(emb.T).reshape(VP, D)
  mean_sum = _sc_bag(emb_rows, idx_o, w_o)
  return _mlp(mean_sum, W1, b1.reshape(1, H), W2, b2.reshape(1, 2))


# speedup vs baseline: 1.1028x; 1.0761x over previous
import functools

import jax
import jax.numpy as jnp
from jax import lax
from jax.experimental import pallas as pl
from jax.experimental.pallas import tpu as pltpu
from jax.experimental.pallas import tpu_sc as plsc

B = 16384
K = 50
V = 1000000
D = 32
H = 250

NC = 2
NS = 16
NW = NC * NS
BPW = B // NW
RB = 2
RIDX = 2 * K + 4
NR = BPW // RB
NBUF = 4
NI = NR // NBUF
KP = 56


def _sc_stage_body(x_hbm, w_hbm, idx_o, w_o, xbuf, wbuf, idx_buf, w_buf2):
  wid = lax.axis_index("s") * NC + lax.axis_index("c")
  base_b = wid * BPW
  pltpu.sync_copy(x_hbm.at[pl.ds(0, K), pl.ds(base_b, BPW)], xbuf)
  pltpu.sync_copy(w_hbm.at[pl.ds(0, K), pl.ds(base_b, BPW)], wbuf)

  iota = lax.iota(jnp.int32, 16)
  pad_base = iota * 1021 + wid * 37

  def zero_pad(r, _):
    idx_buf[r, pl.ds(RIDX - 16, 16)] = pad_base + r * 53
    return 0

  lax.fori_loop(0, NR, zero_pad, 0)
  colpar = (iota & 1) * K

  def fill_k(k, _):
    for b16 in range(BPW // 16):
      rows = 8 * b16 + (iota >> 1)
      cols = colpar + k
      v = xbuf[k, pl.ds(16 * b16, 16)]
      p = ((v & ~(TRB - 1)) | ((v & (TRB // 4 - 1)) << 2)
           | ((v & (TRB - 1)) >> 11))
      plsc.store_scatter(idx_buf, [rows, cols], p)
      w = wbuf[k, pl.ds(16 * b16, 16)]
      plsc.store_scatter(w_buf2, [rows, cols], w)
    return 0

  lax.fori_loop(0, K, fill_k, 0)

  pltpu.sync_copy(idx_buf, idx_o.at[pl.ds(wid * NR, NR), :])
  pltpu.sync_copy(w_buf2, w_o.at[pl.ds(wid * NR, NR), :])


@jax.jit
def _sc_stage(xt_pad, wt_pad):
  mesh = plsc.VectorSubcoreMesh(core_axis_name="c", subcore_axis_name="s",
                                num_cores=NC, num_subcores=NS)
  f = pl.kernel(
      _sc_stage_body,
      out_type=(jax.ShapeDtypeStruct((NW * NR, RIDX), jnp.int32),
                jax.ShapeDtypeStruct((NW * NR, RIDX), jnp.float32)),
      mesh=mesh,
      compiler_params=pltpu.CompilerParams(use_tc_tiling_on_sc=False,
                                           needs_layout_passes=False),
      scratch_types=[
          pltpu.VMEM((K, BPW), jnp.int32),
          pltpu.VMEM((K, BPW), jnp.float32),
          pltpu.VMEM((NR, RIDX), jnp.int32),
          pltpu.VMEM((NR, RIDX), jnp.float32),
      ],
  )
  return f(xt_pad, wt_pad)


def _sc_bag_body(emb_hbm, x_hbm, w_hbm, out_hbm,
                 idx_all, w_all, rows0, rows1, rows2, rows3, out_buf,
                 sem0, sem1, sem2, sem3, sem_i, sem_w):
  wid = lax.axis_index("s") * NC + lax.axis_index("c")
  base_b = wid * BPW
  bufs = (rows0, rows1, rows2, rows3)
  sems = (sem0, sem1, sem2, sem3)

  cp_i = pltpu.async_copy(x_hbm.at[pl.ds(wid * NR, NR), :], idx_all, sem_i)
  cp_w = pltpu.async_copy(w_hbm.at[pl.ds(wid * NR, NR), :], w_all, sem_w)
  cp_i.wait()

  def start_gather(r, buf, sem):
    pltpu.async_copy(emb_hbm.at[idx_all.at[r]], buf, sem)

  for s in range(NBUF):
    start_gather(s, bufs[s], sems[s])
  cp_w.wait()

  lane_consts = [jnp.full((16,), lane, jnp.int32) for lane in range(16)]

  def lane_bcast(vec, lane):
    return lax.gather(
        vec, lane_consts[lane][:, None],
        lax.GatherDimensionNumbers(offset_dims=(), collapsed_slice_dims=(0,),
                                   start_index_map=(0,)),
        (1,), mode=lax.GatherScatterMode.PROMISE_IN_BOUNDS)

  def compute_round(r, rows):
    for j in range(RB):
      b_loc = r * RB + j
      cb = j * K
      wvecs = [w_all[r, pl.ds(cb + off, 16)] for off in (0, 16, 32, 34)]
      acc = [[jnp.zeros((16,), jnp.float32) for _ in range(2)]
             for _ in range(5)]
      for k in range(K):
        g = k % 5
        row = j * K + k
        if k < 48:
          wv = lane_bcast(wvecs[k // 16], k % 16)
        else:
          wv = lane_bcast(wvecs[3], k - 34)
        acc[g][0] = acc[g][0] + wv * rows[row, pl.ds(0, 16)]
        acc[g][1] = acc[g][1] + wv * rows[row, pl.ds(16, 16)]
      lo = ((acc[0][0] + acc[1][0]) + (acc[2][0] + acc[3][0])) + acc[4][0]
      hi = ((acc[0][1] + acc[1][1]) + (acc[2][1] + acc[3][1])) + acc[4][1]
      out_buf[b_loc, pl.ds(0, 16)] = lo
      out_buf[b_loc, pl.ds(16, 16)] = hi

  def loop_body(i, _):
    r0 = i * NBUF
    for s in range(NBUF):
      r = r0 + s
      pltpu.make_async_copy(emb_hbm.at[idx_all.at[r]], bufs[s],
                            sems[s]).wait()
      compute_round(r, bufs[s])

      @pl.when(r + NBUF < NR)
      def _():
        start_gather(r + NBUF, bufs[s], sems[s])

    return 0

  lax.fori_loop(0, NI, loop_body, 0)

  pltpu.sync_copy(out_buf, out_hbm.at[pl.ds(base_b, BPW), :])


@jax.jit
def _sc_bag(emb, x2, w_flat):
  mesh = plsc.VectorSubcoreMesh(core_axis_name="c", subcore_axis_name="s",
                                num_cores=NC, num_subcores=NS)
  f = pl.kernel(
      _sc_bag_body,
      out_type=jax.ShapeDtypeStruct((B, D), jnp.float32),
      mesh=mesh,
      compiler_params=pltpu.CompilerParams(use_tc_tiling_on_sc=False),
      scratch_types=[
          pltpu.VMEM((NR, RIDX), jnp.int32),
          pltpu.VMEM((NR, RIDX), jnp.float32),
          pltpu.VMEM((RIDX, D), jnp.float32),
          pltpu.VMEM((RIDX, D), jnp.float32),
          pltpu.VMEM((RIDX, D), jnp.float32),
          pltpu.VMEM((RIDX, D), jnp.float32),
          pltpu.VMEM((BPW, D), jnp.float32),
          pltpu.SemaphoreType.DMA,
          pltpu.SemaphoreType.DMA,
          pltpu.SemaphoreType.DMA,
          pltpu.SemaphoreType.DMA,
          pltpu.SemaphoreType.DMA,
          pltpu.SemaphoreType.DMA,
      ],
  )
  return f(emb, x2, w_flat)


TRB = 8192
TRBM = 3
NTRB = (V + TRB - 1) // TRB
VP = NTRB * TRB


def _tr_body(in_ref, out_ref):
  for m in range(TRBM):
    parts = [in_ref[:, pl.ds(m * TRB + a * (TRB // 4), TRB // 4)]
             for a in range(4)]
    out_ref[pl.ds(m * (TRB // 4), TRB // 4), :] = (
        jnp.concatenate(parts, axis=0).T)


@jax.jit
def _relayout(emb_t):
  assert NTRB % TRBM == 0
  return pl.pallas_call(
      _tr_body,
      grid=(NTRB // TRBM,),
      in_specs=[pl.BlockSpec((D, TRBM * TRB), lambda i: (0, i))],
      out_specs=pl.BlockSpec((TRBM * TRB // 4, 4 * D), lambda i: (i, 0)),
      out_shape=jax.ShapeDtypeStruct((VP // 4, 4 * D), jnp.float32),
  )(emb_t)


def _mlp_body(mean_ref, w1_ref, b1_ref, w2_ref, b2_ref, out_ref):
  w1 = w1_ref[:] * (1.0 / K)
  h = jnp.dot(mean_ref[:], w1, preferred_element_type=jnp.float32)
  h = jnp.maximum(h + b1_ref[:], 0.0)
  out_ref[:] = jnp.dot(h, w2_ref[:], preferred_element_type=jnp.float32) \
      + b2_ref[:]


@jax.jit
def _mlp(mean, W1, b1, W2, b2):
  M = 2048
  grid = (B // M,)
  return pl.pallas_call(
      _mlp_body,
      grid=grid,
      in_specs=[
          pl.BlockSpec((M, D), lambda i: (i, 0)),
          pl.BlockSpec((D, H), lambda i: (0, 0)),
          pl.BlockSpec((1, H), lambda i: (0, 0)),
          pl.BlockSpec((H, 2), lambda i: (0, 0)),
          pl.BlockSpec((1, 2), lambda i: (0, 0)),
      ],
      out_specs=pl.BlockSpec((M, 2), lambda i: (i, 0)),
      out_shape=jax.ShapeDtypeStruct((B, 2), jnp.float32),
  )(mean, W1, b1, W2, b2)


def kernel(x, T, emb, W1, b1, W2, b2):
  xt_pad = jnp.pad(x.astype(jnp.int32).T, ((0, KP - K), (0, 0)))
  wt_pad = jnp.pad(jnp.transpose(T, (2, 1, 0)).reshape(K, B),
                   ((0, KP - K), (0, 0)))
  idx_o, w_o = _sc_stage(xt_pad, wt_pad)
  emb_rows = _relayout(emb.T).reshape(VP, D)
  mean_sum = _sc_bag(emb_rows, idx_o, w_o)
  return _mlp(mean_sum, W1, b1.reshape(1, H), W2, b2.reshape(1, 2))
